# baseline (speedup 1.0000x reference)
import functools

import jax
import jax.numpy as jnp
from jax import lax
from jax.experimental import pallas as pl
from jax.experimental.pallas import tpu as pltpu
import jax.experimental.pallas.tpu_sc as plsc

N = 320000
D = 128
S = 2048
NC = 2
NS = 16
NW = NC * NS
SPT = S // NW
SPC = S // NC
C = 128
NBUF = 6
PD = 4
NSTREAM = 4
ACC_ROWS = SPT + 8
SH_ROWS = SPC + 8
LANES = 16
UNROLL = 4


def _tile_body(nodes_hbm, batch_hbm, bounds_hbm, out_hbm,
               rows_v, ids_v, lidx_v, bnd_v, acc_v, mbuf_v, acc_sh, *sems):
    rsems = sems[:NBUF]
    isems = sems[NBUF:2 * NBUF]
    ssems = sems[2 * NBUF:]
    sid = lax.axis_index("s")
    cid = lax.axis_index("c")
    wid = cid * NS + sid
    seg_base = wid * SPT
    sh_base = sid * SPT
    lanes = lax.iota(jnp.int32, 16)

    pltpu.sync_copy(bounds_hbm.at[pl.ds(wid, 1)], bnd_v)
    bvec = bnd_v[0, :]
    start = bvec[0]
    end = bvec[1]

    zz = jnp.zeros((LANES,), jnp.float32)

    def zero_row(i, carry):
        for j in range(D // LANES):
            acc_v[i, pl.ds(j * LANES, LANES)] = zz
        return carry

    lax.fori_loop(0, ACC_ROWS, zero_row, 0)
    pltpu.sync_copy(acc_v.at[pl.ds(0, SPT)],
                    acc_sh.at[pl.ds(sh_base, SPT)])

    @pl.when(sid == 0)
    def _():
        pltpu.sync_copy(acc_v.at[pl.ds(0, SH_ROWS - SPC)],
                        acc_sh.at[pl.ds(SPC, SH_ROWS - SPC)])

    plsc.subcore_barrier()

    astart = jnp.bitwise_and(start, jnp.int32(-8))
    nchunks = lax.shift_right_arithmetic(end - astart + (C - 1), 7)
    nouter = (nchunks + (NBUF - 1)) // NBUF

    def chunk_base(k):
        return pl.multiple_of(jnp.minimum(astart + k * C, N - C), 8)

    def fetch(k, b):
        base = chunk_base(k)
        pltpu.async_copy(batch_hbm.at[pl.ds(base, C)], ids_v.at[b], isems[b])
        pltpu.async_copy(nodes_hbm.at[pl.ds(base, C)], rows_v.at[b], rsems[b])

    def wait_fetch(k, b):
        base = chunk_base(k)
        pltpu.make_async_copy(
            batch_hbm.at[pl.ds(base, C)], ids_v.at[b], isems[b]).wait()
        pltpu.make_async_copy(
            nodes_hbm.at[pl.ds(base, C)], rows_v.at[b], rsems[b]).wait()

    def wait_scatter(b):
        pltpu.make_async_copy(
            rows_v.at[b], acc_sh.at[lidx_v.at[b]], ssems[b]).wait()

    def valu_chunk(b):
        def vgroup(g, carry):
            idvec = lidx_v[b, pl.ds(g * LANES, LANES)]
            for l in range(LANES):
                r = g * LANES + l
                sidx = idvec[l]
                for j in range(D // LANES):
                    sl = pl.ds(j * LANES, LANES)
                    acc_v[sidx, sl] = acc_v[sidx, sl] + rows_v[b, r, sl]
            return carry

        lax.fori_loop(0, C // LANES, vgroup, 0)

    for b in range(PD):
        fetch(jnp.int32(b), b)

    def outer(k0, carry):
        for b in range(NBUF):
            k = k0 * NBUF + b
            stream = b < NSTREAM
            nominal = astart + k * C
            lo = jnp.maximum(start, nominal)
            hi = jnp.minimum(end, nominal + C)
            base = chunk_base(k)
            wait_fetch(k, b)
            dummy = jnp.int32(SPC if stream else SPT)
            off = cid * SPC if stream else seg_base
            for g in range(C // LANES):
                rg = base + (g * LANES) + lanes
                idv = ids_v[b, pl.ds(g * LANES, LANES)]
                keep = ((rg >= lo) & (rg < hi)
                        & (idv >= seg_base) & (idv < seg_base + SPT))
                lidx_v[b, pl.ds(g * LANES, LANES)] = jnp.where(
                    keep, idv - off, dummy)
            if stream:
                pltpu.async_copy(rows_v.at[b], acc_sh.at[lidx_v.at[b]],
                                 ssems[b], add=True)
            else:
                valu_chunk(b)
            b2 = (b + PD) % NBUF
            if b2 < NSTREAM:
                wait_scatter(b2)
            fetch(k + PD, b2)
        return carry

    lax.fori_loop(0, nouter, outer, 0)

    T = nouter * NBUF
    for d in range(PD):
        wait_fetch(T + d, d)

    pltpu.sync_copy(acc_sh.at[pl.ds(sh_base, SPT)], mbuf_v)

    def merge_row(i, carry):
        for j in range(D // LANES):
            sl = pl.ds(j * LANES, LANES)
            acc_v[i, sl] = acc_v[i, sl] + mbuf_v[i, sl]
        return carry

    lax.fori_loop(0, SPT, merge_row, 0)
    pltpu.sync_copy(acc_v.at[pl.ds(0, SPT)],
                    out_hbm.at[pl.ds(seg_base, SPT)])


@functools.partial(
    pl.kernel,
    out_type=jax.ShapeDtypeStruct((S, D), jnp.float32),
    mesh=plsc.VectorSubcoreMesh(core_axis_name="c", subcore_axis_name="s"),
    scratch_types=[
        pltpu.VMEM((NBUF, C, D), jnp.float32),
        pltpu.VMEM((NBUF, C), jnp.int32),
        pltpu.VMEM((NBUF, C), jnp.int32),
        pltpu.VMEM((1, 16), jnp.int32),
        pltpu.VMEM((ACC_ROWS, D), jnp.float32),
        pltpu.VMEM((SPT, D), jnp.float32),
        pltpu.MemorySpace.VMEM_SHARED((SH_ROWS, D), jnp.float32),
    ] + [pltpu.SemaphoreType.DMA] * (3 * NBUF),
)
def _segment_sum_sc(nodes_hbm, batch_hbm, bounds_hbm, out_hbm,
                    rows_v, ids_v, lidx_v, bnd_v, acc_v, mbuf_v, acc_sh,
                    *sems):
    _tile_body(nodes_hbm, batch_hbm, bounds_hbm, out_hbm,
               rows_v, ids_v, lidx_v, bnd_v, acc_v, mbuf_v, acc_sh, *sems)


def kernel(node_features, batch, ptr):
    edges = jnp.arange(0, S + 1, SPT, dtype=jnp.int32)
    starts = jnp.searchsorted(batch, edges, side="left").astype(jnp.int32)
    bounds = jnp.zeros((NW, 16), jnp.int32)
    bounds = bounds.at[:, 0].set(starts[:-1]).at[:, 1].set(starts[1:])
    return _segment_sum_sc(node_features, batch, bounds)


# speedup vs baseline: 2.0789x; 2.0789x over previous
import functools

import jax
import jax.numpy as jnp
from jax import lax
from jax.experimental import pallas as pl
from jax.experimental.pallas import tpu as pltpu
import jax.experimental.pallas.tpu_sc as plsc

N = 320000
D = 128
S = 2048
NC = 2
NS = 16
NW = NC * NS
RPT = N // NW
C = 80
NCHUNKS = RPT // C
NBUF = 5
PD = 3
NOUTER = NCHUNKS // NBUF
SROWS = S // NS
LANES = 16


def _tile_body(nodes_hbm, batch2d_hbm, pout_hbm,
               rows_v, ids_v, zbuf_v, acc_sh, *sems):
    rsems = sems[:NBUF]
    ssems = sems[NBUF:]
    sid = lax.axis_index("s")
    cid = lax.axis_index("c")
    wid = cid * NS + sid
    row0 = wid * RPT

    def chunk_base(k):
        return pl.multiple_of(jnp.minimum(row0 + k * C, N - C), 8)

    def fetch(k, b):
        pltpu.async_copy(nodes_hbm.at[pl.ds(chunk_base(k), C)],
                         rows_v.at[b], rsems[b])

    def wait_fetch(k, b):
        pltpu.make_async_copy(
            nodes_hbm.at[pl.ds(chunk_base(k), C)],
            rows_v.at[b], rsems[b]).wait()

    def scatter(k, b):
        pltpu.async_copy(rows_v.at[b], acc_sh.at[ids_v.at[k]],
                         ssems[b], add=True)

    def wait_scatter(k, b):
        pltpu.make_async_copy(
            rows_v.at[b], acc_sh.at[ids_v.at[k]], ssems[b]).wait()

    pltpu.sync_copy(batch2d_hbm.at[wid], ids_v)
    for b in range(PD):
        fetch(jnp.int32(b), b)

    zz = jnp.zeros((LANES,), jnp.float32)

    def zero_row(i, carry):
        for j in range(D // LANES):
            zbuf_v[i, pl.ds(j * LANES, LANES)] = zz
        return carry

    lax.fori_loop(0, SROWS, zero_row, 0)
    pltpu.sync_copy(zbuf_v, acc_sh.at[pl.ds(sid * SROWS, SROWS)])
    plsc.subcore_barrier()

    def outer(k0, carry):
        for b in range(NBUF):
            k = k0 * NBUF + b
            wait_fetch(k, b)
            b2 = (b + PD) % NBUF
            if b in (0, 1):
                @pl.when(k0 > 0)
                def _():
                    wait_scatter(k - 2, b2)
            else:
                wait_scatter(k - 2, b2)
            scatter(k, b)
            fetch(k + PD, b2)
        return carry

    lax.fori_loop(0, NOUTER, outer, 0)

    for b in range(PD):
        wait_fetch(NCHUNKS + b, b)
    wait_scatter(NCHUNKS - 2, NBUF - 2)
    wait_scatter(NCHUNKS - 1, NBUF - 1)

    plsc.subcore_barrier()
    pltpu.sync_copy(acc_sh.at[pl.ds(sid * SROWS, SROWS)],
                    pout_hbm.at[cid].at[pl.ds(sid * SROWS, SROWS)])


@functools.partial(
    pl.kernel,
    out_type=jax.ShapeDtypeStruct((NC, S, D), jnp.float32),
    mesh=plsc.VectorSubcoreMesh(core_axis_name="c", subcore_axis_name="s"),
    scratch_types=[
        pltpu.VMEM((NBUF, C, D), jnp.float32),
        pltpu.VMEM((NCHUNKS, C), jnp.int32),
        pltpu.VMEM((SROWS, D), jnp.float32),
        pltpu.MemorySpace.VMEM_SHARED((S, D), jnp.float32),
    ] + [pltpu.SemaphoreType.DMA] * (2 * NBUF),
)
def _segment_sum_sc(nodes_hbm, batch2d_hbm, pout_hbm,
                    rows_v, ids_v, zbuf_v, acc_sh, *sems):
    _tile_body(nodes_hbm, batch2d_hbm, pout_hbm,
               rows_v, ids_v, zbuf_v, acc_sh, *sems)


def _combine_body(p_ref, o_ref):
    o_ref[...] = p_ref[0] + p_ref[1]


def _combine(partials):
    blk = 256
    return pl.pallas_call(
        _combine_body,
        grid=(S // blk,),
        in_specs=[pl.BlockSpec((NC, blk, D), lambda i: (0, i, 0))],
        out_specs=pl.BlockSpec((blk, D), lambda i: (i, 0)),
        out_shape=jax.ShapeDtypeStruct((S, D), jnp.float32),
    )(partials)


def kernel(node_features, batch, ptr):
    partials = _segment_sum_sc(node_features, batch.reshape(NW, NCHUNKS, C))
    return _combine(partials)
